# baseline (speedup 1.0000x reference)
import jax
import jax.numpy as jnp
from jax import lax
from jax.experimental import pallas as pl
from jax.experimental.pallas import tpu as pltpu
from jax.experimental.pallas import tpu_sc as plsc

N_NODES = 10000
N_EDGES = 320000
HALF = N_NODES // 2
APAD = 5120
ARPT = APAD // 16
NTRASH = APAD - HALF
CHUNK = 80
EPT = N_EDGES // 32
NCHUNK = EPT // CHUNK
NCKP = 128


def _make_sc_agg(nrows, passes):
    mesh = plsc.VectorSubcoreMesh(core_axis_name="c", subcore_axis_name="s")
    npass = len(passes)

    def body(z_hbm, src_hbm, dst_hbm, bnd_hbm, zero_hbm, out_hbm,
             srcv, dstv, bndv, buf0, buf1, accum, gsem0, gsem1):
        c = lax.axis_index("c")
        s = lax.axis_index("s")
        w = c * 16 + s
        for pi, (q, f) in enumerate(passes):
            pltpu.sync_copy(src_hbm.at[f * 32 + w], srcv)
            pltpu.sync_copy(dst_hbm.at[q * 32 + w], dstv)
            pltpu.sync_copy(bnd_hbm.at[q * 32 + w], bndv)
            pltpu.sync_copy(zero_hbm, accum.at[pl.ds(s * ARPT, ARPT)])
            plsc.subcore_barrier()
            bv = bndv[...]
            lo = bv[0]
            npair = bv[1]

            @pl.when(npair > 0)
            def _():
                pltpu.async_copy(z_hbm.at[srcv.at[lo]], buf0, gsem0)

            def pair(i, carry):
                jo = lo + 2 * i
                pltpu.make_async_copy(
                    z_hbm.at[srcv.at[jo]], buf0, gsem0).wait()
                pltpu.async_copy(z_hbm.at[srcv.at[jo + 1]], buf1, gsem1)
                pltpu.sync_copy(buf0, accum.at[dstv.at[jo]], add=True)
                pltpu.make_async_copy(
                    z_hbm.at[srcv.at[jo + 1]], buf1, gsem1).wait()

                @pl.when(i + 1 < npair)
                def _():
                    pltpu.async_copy(z_hbm.at[srcv.at[jo + 2]], buf0, gsem0)

                pltpu.sync_copy(buf1, accum.at[dstv.at[jo + 1]], add=True)
                return carry

            lax.fori_loop(0, npair, pair, 0)
            plsc.subcore_barrier()
            pltpu.sync_copy(
                accum.at[pl.ds(s * ARPT, ARPT)],
                out_hbm.at[pl.ds((c * npass + pi) * APAD + s * ARPT, ARPT)])

    return pl.kernel(
        body,
        out_type=jax.ShapeDtypeStruct((2 * npass * APAD, 128), jnp.float32),
        mesh=mesh,
        scratch_types=[
            pltpu.VMEM((NCKP, CHUNK), jnp.int32),
            pltpu.VMEM((NCKP, CHUNK), jnp.int32),
            pltpu.VMEM((16,), jnp.int32),
            pltpu.VMEM((CHUNK, 128), jnp.float32),
            pltpu.VMEM((CHUNK, 128), jnp.float32),
            pltpu.VMEM_SHARED((APAD, 128), jnp.float32),
            pltpu.SemaphoreType.DMA,
            pltpu.SemaphoreType.DMA,
        ],
    )


def _make_mlp_body(npass):
    def body(z_ref, agg_ref, w1_ref, b1_ref, w2_ref, b2_ref, g_ref, bt_ref,
             out_ref):
        def piece(c, q, f):
            base = (c * npass + q * (npass // 2) + f) * APAD
            return agg_ref[base:base + HALF, :]

        halves = []
        for q in (0, 1):
            if npass == 2:
                agg_q = piece(0, q, 0) + piece(1, q, 0)
            else:
                agg_q = jnp.concatenate(
                    [piece(0, q, 0) + piece(1, q, 0),
                     piece(0, q, 1) + piece(1, q, 1)], axis=1)
            halves.append(z_ref[q * HALF:(q + 1) * HALF, :] + agg_q)
        h = jnp.concatenate(halves, axis=0)
        h = jnp.maximum(
            jnp.dot(h, w1_ref[...], preferred_element_type=jnp.float32)
            + b1_ref[...], 0.0)
        h = jnp.maximum(
            jnp.dot(h, w2_ref[...], preferred_element_type=jnp.float32)
            + b2_ref[...], 0.0)
        mu = jnp.mean(h, axis=0, keepdims=True)
        var = jnp.mean(h * h, axis=0, keepdims=True) - mu * mu
        out_ref[...] = ((h - mu) * lax.rsqrt(var + 1e-5) * g_ref[...]
                        + bt_ref[...])
    return body


def _mlp(z, agg, p, npass):
    hid = p['W1'].shape[1]
    return pl.pallas_call(
        _make_mlp_body(npass),
        out_shape=jax.ShapeDtypeStruct((z.shape[0], hid), jnp.float32),
    )(z, agg, p['W1'], p['b1'].reshape(1, -1), p['W2'],
      p['b2'].reshape(1, -1), p['gamma'].reshape(1, -1),
      p['beta'].reshape(1, -1))


def _pad_chunks(a, fill):
    a = a.reshape(32, NCHUNK, CHUNK)
    pad = jnp.full((32, NCKP - NCHUNK, CHUNK), fill, jnp.int32)
    return jnp.concatenate([a, pad], axis=1)


def kernel(x, edge_index, edge_weight, params):
    src = edge_index[0].astype(jnp.int32)
    dst = edge_index[1].astype(jnp.int32)

    key = (dst >= HALF).astype(jnp.int32)
    c1 = jnp.cumsum(key)
    cut0 = N_EDGES - c1[-1]
    e_ix = jnp.arange(N_EDGES, dtype=jnp.int32)
    pos = jnp.where(key == 1, cut0 + c1 - 1, e_ix - c1)
    src = jnp.zeros((N_EDGES,), jnp.int32).at[pos].set(src)
    dst = jnp.zeros((N_EDGES,), jnp.int32).at[pos].set(dst)

    trash = HALF + (jnp.arange(N_EDGES, dtype=jnp.int32) % NTRASH)
    live = src != dst
    dst_qs = []
    for q in (0, 1):
        in_half = live & (dst >= q * HALF) & (dst < (q + 1) * HALF)
        dst_qs.append(_pad_chunks(jnp.where(in_half, dst - q * HALF, trash),
                                  HALF))
    dsts = jnp.concatenate(dst_qs, axis=0)

    srcs1 = _pad_chunks(src, 0)
    srcs2 = jnp.concatenate([srcs1, srcs1 + N_NODES], axis=0)

    cut = N_EDGES - jnp.sum(key)
    hi0 = jnp.clip(cut - jnp.arange(32) * EPT, 0, EPT).astype(jnp.int32)
    nhi = (hi0 + CHUNK - 1) // CHUNK
    flo = hi0 // CHUNK
    np0 = (nhi + 1) // 2
    np1 = (NCHUNK - flo + 1) // 2
    bounds = jnp.zeros((64, 16), jnp.int32)
    bounds = bounds.at[:32, 1].set(np0)
    bounds = bounds.at[32:, 0].set(flo)
    bounds = bounds.at[32:, 1].set(np1)

    zeros = jnp.zeros((ARPT, 128), jnp.float32)
    agg2 = _make_sc_agg(N_NODES, [(0, 0), (1, 0)])
    agg4 = _make_sc_agg(2 * N_NODES, [(0, 0), (0, 1), (1, 0), (1, 1)])

    z = x
    outs = []
    for p in params:
        d = z.shape[1]
        if d == 128:
            agg = agg2(z, srcs1, dsts, bounds, zeros)
            z = _mlp(z, agg, p, 2)
        else:
            dh = d // 2
            z_stack = jnp.concatenate([z[:, :dh], z[:, dh:]], axis=0)
            agg = agg4(z_stack, srcs2, dsts, bounds, zeros)
            z = _mlp(z, agg, p, 4)
        outs.append(z)
    return jnp.concatenate(outs, axis=1)


# speedup vs baseline: 1.4923x; 1.4923x over previous
import jax
import jax.numpy as jnp
from jax import lax
from jax.experimental import pallas as pl
from jax.experimental.pallas import tpu as pltpu
from jax.experimental.pallas import tpu_sc as plsc

N_NODES = 10000
N_EDGES = 320000
HALF = N_NODES // 2
APAD = 5120
ARPT = APAD // 16
NTRASH = APAD - HALF
CHUNK = 80
EPT = N_EDGES // 32
NCHUNK = EPT // CHUNK
NCKP = 128


def _make_sc_agg(nrows, passes):
    mesh = plsc.VectorSubcoreMesh(core_axis_name="c", subcore_axis_name="s")
    npass = len(passes)

    def body(z_hbm, src_hbm, dst_hbm, bnd_hbm, zero_hbm, out_hbm,
             srcv, dstv, bndv, b0, b1, b2, b3, accum,
             g0, g1, g2, g3, s0, s1, s2, s3):
        bufs = (b0, b1, b2, b3)
        gsems = (g0, g1, g2, g3)
        ssems = (s0, s1, s2, s3)
        c = lax.axis_index("c")
        s = lax.axis_index("s")
        w = c * 16 + s
        for pi, (q, f) in enumerate(passes):
            pltpu.sync_copy(src_hbm.at[f * 32 + w], srcv)
            pltpu.sync_copy(dst_hbm.at[q * 32 + w], dstv)
            pltpu.sync_copy(bnd_hbm.at[q * 32 + w], bndv)
            pltpu.sync_copy(zero_hbm, accum.at[pl.ds(s * ARPT, ARPT)])
            plsc.subcore_barrier()
            bv = bndv[...]
            lo = bv[0]
            nquad = bv[1]

            def gather(j, b):
                pltpu.async_copy(z_hbm.at[srcv.at[j]], bufs[b], gsems[b])

            def gather_wait(j, b):
                pltpu.make_async_copy(
                    z_hbm.at[srcv.at[j]], bufs[b], gsems[b]).wait()

            def scat(j, b):
                pltpu.async_copy(bufs[b], accum.at[dstv.at[j]], ssems[b],
                                 add=True)

            def scat_wait(j, b):
                pltpu.make_async_copy(
                    bufs[b], accum.at[dstv.at[j]], ssems[b]).wait()

            @pl.when(nquad > 0)
            def _():
                gather(lo, 0)
                gather(lo + 1, 1)
                gather(lo + 2, 2)

            def quad(i, carry):
                j = lo + 4 * i

                @pl.when(i > 0)
                def _():
                    scat_wait(j - 1, 3)

                gather(j + 3, 3)
                gather_wait(j, 0)
                scat(j, 0)
                gather_wait(j + 1, 1)
                scat(j + 1, 1)
                scat_wait(j, 0)

                @pl.when(i + 1 < nquad)
                def _():
                    gather(j + 4, 0)

                gather_wait(j + 2, 2)
                scat(j + 2, 2)
                scat_wait(j + 1, 1)

                @pl.when(i + 1 < nquad)
                def _():
                    gather(j + 5, 1)

                gather_wait(j + 3, 3)
                scat(j + 3, 3)
                scat_wait(j + 2, 2)

                @pl.when(i + 1 < nquad)
                def _():
                    gather(j + 6, 2)

                return carry

            lax.fori_loop(0, nquad, quad, 0)

            @pl.when(nquad > 0)
            def _():
                scat_wait(lo + 4 * nquad - 1, 3)

            plsc.subcore_barrier()
            pltpu.sync_copy(
                accum.at[pl.ds(s * ARPT, ARPT)],
                out_hbm.at[pl.ds((c * npass + pi) * APAD + s * ARPT, ARPT)])

    return pl.kernel(
        body,
        out_type=jax.ShapeDtypeStruct((2 * npass * APAD, 128), jnp.float32),
        mesh=mesh,
        scratch_types=[
            pltpu.VMEM((NCKP, CHUNK), jnp.int32),
            pltpu.VMEM((NCKP, CHUNK), jnp.int32),
            pltpu.VMEM((16,), jnp.int32),
            pltpu.VMEM((CHUNK, 128), jnp.float32),
            pltpu.VMEM((CHUNK, 128), jnp.float32),
            pltpu.VMEM((CHUNK, 128), jnp.float32),
            pltpu.VMEM((CHUNK, 128), jnp.float32),
            pltpu.VMEM_SHARED((APAD, 128), jnp.float32),
        ] + [pltpu.SemaphoreType.DMA] * 8,
    )


def _make_mlp_body(npass):
    def body(z_ref, agg_ref, w1_ref, b1_ref, w2_ref, b2_ref, g_ref, bt_ref,
             out_ref):
        def piece(c, q, f):
            base = (c * npass + q * (npass // 2) + f) * APAD
            return agg_ref[base:base + HALF, :]

        halves = []
        for q in (0, 1):
            if npass == 2:
                agg_q = piece(0, q, 0) + piece(1, q, 0)
            else:
                agg_q = jnp.concatenate(
                    [piece(0, q, 0) + piece(1, q, 0),
                     piece(0, q, 1) + piece(1, q, 1)], axis=1)
            halves.append(z_ref[q * HALF:(q + 1) * HALF, :] + agg_q)
        h = jnp.concatenate(halves, axis=0)
        h = jnp.maximum(
            jnp.dot(h, w1_ref[...], preferred_element_type=jnp.float32)
            + b1_ref[...], 0.0)
        h = jnp.maximum(
            jnp.dot(h, w2_ref[...], preferred_element_type=jnp.float32)
            + b2_ref[...], 0.0)
        mu = jnp.mean(h, axis=0, keepdims=True)
        var = jnp.mean(h * h, axis=0, keepdims=True) - mu * mu
        out_ref[...] = ((h - mu) * lax.rsqrt(var + 1e-5) * g_ref[...]
                        + bt_ref[...])
    return body


def _mlp(z, agg, p, npass):
    hid = p['W1'].shape[1]
    return pl.pallas_call(
        _make_mlp_body(npass),
        out_shape=jax.ShapeDtypeStruct((z.shape[0], hid), jnp.float32),
    )(z, agg, p['W1'], p['b1'].reshape(1, -1), p['W2'],
      p['b2'].reshape(1, -1), p['gamma'].reshape(1, -1),
      p['beta'].reshape(1, -1))


def _pad_chunks(a, fill):
    a = a.reshape(32, NCHUNK, CHUNK)
    pad = jnp.full((32, NCKP - NCHUNK, CHUNK), fill, jnp.int32)
    return jnp.concatenate([a, pad], axis=1)


def kernel(x, edge_index, edge_weight, params):
    src = edge_index[0].astype(jnp.int32)
    dst = edge_index[1].astype(jnp.int32)

    key = (dst >= HALF).astype(jnp.int32)
    perm = jnp.argsort(key)
    src = src[perm]
    dst = dst[perm]

    trash = HALF + (jnp.arange(N_EDGES, dtype=jnp.int32) % NTRASH)
    live = src != dst
    dst_qs = []
    for q in (0, 1):
        in_half = live & (dst >= q * HALF) & (dst < (q + 1) * HALF)
        dst_qs.append(_pad_chunks(jnp.where(in_half, dst - q * HALF, trash),
                                  HALF))
    dsts = jnp.concatenate(dst_qs, axis=0)

    srcs1 = _pad_chunks(src, 0)
    srcs2 = jnp.concatenate([srcs1, srcs1 + N_NODES], axis=0)

    cut = N_EDGES - jnp.sum(key)
    hi0 = jnp.clip(cut - jnp.arange(32) * EPT, 0, EPT).astype(jnp.int32)
    nhi = (hi0 + CHUNK - 1) // CHUNK
    flo = hi0 // CHUNK
    np0 = (nhi + 3) // 4
    np1 = (NCHUNK - flo + 3) // 4
    bounds = jnp.zeros((64, 16), jnp.int32)
    bounds = bounds.at[:32, 1].set(np0)
    bounds = bounds.at[32:, 0].set(flo)
    bounds = bounds.at[32:, 1].set(np1)

    zeros = jnp.zeros((ARPT, 128), jnp.float32)
    agg2 = _make_sc_agg(N_NODES, [(0, 0), (1, 0)])
    agg4 = _make_sc_agg(2 * N_NODES, [(0, 0), (0, 1), (1, 0), (1, 1)])

    z = x
    outs = []
    for p in params:
        d = z.shape[1]
        if d == 128:
            agg = agg2(z, srcs1, dsts, bounds, zeros)
            z = _mlp(z, agg, p, 2)
        else:
            dh = d // 2
            z_stack = jnp.concatenate([z[:, :dh], z[:, dh:]], axis=0)
            agg = agg4(z_stack, srcs2, dsts, bounds, zeros)
            z = _mlp(z, agg, p, 4)
        outs.append(z)
    return jnp.concatenate(outs, axis=1)


# speedup vs baseline: 2.0960x; 1.4046x over previous
import jax
import jax.numpy as jnp
from jax import lax
from jax.experimental import pallas as pl
from jax.experimental.pallas import tpu as pltpu
from jax.experimental.pallas import tpu_sc as plsc

N_NODES = 10000
N_EDGES = 320000
HALF = N_NODES // 2
APAD = 5120
ARPT = APAD // 16
NTRASH = APAD - HALF
CHUNK = 80
EPT = N_EDGES // 32
NCHUNK = EPT // CHUNK
NCKP = 128


def _make_sc_agg(nrows, passes):
    mesh = plsc.VectorSubcoreMesh(core_axis_name="c", subcore_axis_name="s")
    npass = len(passes)

    def body(z_hbm, src_hbm, dst_hbm, bnd_hbm, zero_hbm, out_hbm,
             srcv, dstv, bndv, buf0, buf1, accum, gsem0, gsem1):
        c = lax.axis_index("c")
        s = lax.axis_index("s")
        w = c * 16 + s
        for pi, (q, f) in enumerate(passes):
            pltpu.sync_copy(src_hbm.at[f * 32 + w], srcv)
            pltpu.sync_copy(dst_hbm.at[q * 32 + w], dstv)
            pltpu.sync_copy(bnd_hbm.at[q * 32 + w], bndv)
            pltpu.sync_copy(zero_hbm, accum.at[pl.ds(s * ARPT, ARPT)])
            plsc.subcore_barrier()
            bv = bndv[...]
            lo = bv[0]
            npair = bv[1]

            @pl.when(npair > 0)
            def _():
                pltpu.async_copy(z_hbm.at[srcv.at[lo]], buf0, gsem0)

            def pair(i, carry):
                jo = lo + 2 * i
                pltpu.make_async_copy(
                    z_hbm.at[srcv.at[jo]], buf0, gsem0).wait()
                pltpu.async_copy(z_hbm.at[srcv.at[jo + 1]], buf1, gsem1)
                pltpu.sync_copy(buf0, accum.at[dstv.at[jo]], add=True)
                pltpu.make_async_copy(
                    z_hbm.at[srcv.at[jo + 1]], buf1, gsem1).wait()

                @pl.when(i + 1 < npair)
                def _():
                    pltpu.async_copy(z_hbm.at[srcv.at[jo + 2]], buf0, gsem0)

                pltpu.sync_copy(buf1, accum.at[dstv.at[jo + 1]], add=True)
                return carry

            lax.fori_loop(0, npair, pair, 0)
            plsc.subcore_barrier()
            pltpu.sync_copy(
                accum.at[pl.ds(s * ARPT, ARPT)],
                out_hbm.at[pl.ds((c * npass + pi) * APAD + s * ARPT, ARPT)])

    return pl.kernel(
        body,
        out_type=jax.ShapeDtypeStruct((2 * npass * APAD, 128), jnp.float32),
        mesh=mesh,
        scratch_types=[
            pltpu.VMEM((NCKP, CHUNK), jnp.int32),
            pltpu.VMEM((NCKP, CHUNK), jnp.int32),
            pltpu.VMEM((16,), jnp.int32),
            pltpu.VMEM((CHUNK, 128), jnp.float32),
            pltpu.VMEM((CHUNK, 128), jnp.float32),
            pltpu.VMEM_SHARED((APAD, 128), jnp.float32),
            pltpu.SemaphoreType.DMA,
            pltpu.SemaphoreType.DMA,
        ],
    )


def _make_mlp_body(npass):
    def body(z_ref, agg_ref, w1_ref, b1_ref, w2_ref, b2_ref, g_ref, bt_ref,
             out_ref):
        def piece(c, q, f):
            base = (c * npass + q * (npass // 2) + f) * APAD
            return agg_ref[base:base + HALF, :]

        halves = []
        for q in (0, 1):
            if npass == 2:
                agg_q = piece(0, q, 0) + piece(1, q, 0)
            else:
                agg_q = jnp.concatenate(
                    [piece(0, q, 0) + piece(1, q, 0),
                     piece(0, q, 1) + piece(1, q, 1)], axis=1)
            halves.append(z_ref[q * HALF:(q + 1) * HALF, :] + agg_q)
        h = jnp.concatenate(halves, axis=0)
        h = jnp.maximum(
            jnp.dot(h, w1_ref[...], preferred_element_type=jnp.float32)
            + b1_ref[...], 0.0)
        h = jnp.maximum(
            jnp.dot(h, w2_ref[...], preferred_element_type=jnp.float32)
            + b2_ref[...], 0.0)
        mu = jnp.mean(h, axis=0, keepdims=True)
        var = jnp.mean(h * h, axis=0, keepdims=True) - mu * mu
        out_ref[...] = ((h - mu) * lax.rsqrt(var + 1e-5) * g_ref[...]
                        + bt_ref[...])
    return body


def _mlp(z, agg, p, npass):
    hid = p['W1'].shape[1]
    return pl.pallas_call(
        _make_mlp_body(npass),
        out_shape=jax.ShapeDtypeStruct((z.shape[0], hid), jnp.float32),
    )(z, agg, p['W1'], p['b1'].reshape(1, -1), p['W2'],
      p['b2'].reshape(1, -1), p['gamma'].reshape(1, -1),
      p['beta'].reshape(1, -1))


def _pad_chunks(a, fill):
    a = a.reshape(32, NCHUNK, CHUNK)
    pad = jnp.full((32, NCKP - NCHUNK, CHUNK), fill, jnp.int32)
    return jnp.concatenate([a, pad], axis=1)


def kernel(x, edge_index, edge_weight, params):
    src = edge_index[0].astype(jnp.int32)
    dst = edge_index[1].astype(jnp.int32)

    key = (dst >= HALF).astype(jnp.int32)
    perm = jnp.argsort(key)
    src = src[perm]
    dst = dst[perm]

    trash = HALF + (jnp.arange(N_EDGES, dtype=jnp.int32) % NTRASH)
    live = src != dst
    dst_qs = []
    for q in (0, 1):
        in_half = live & (dst >= q * HALF) & (dst < (q + 1) * HALF)
        dst_qs.append(_pad_chunks(jnp.where(in_half, dst - q * HALF, trash),
                                  HALF))
    dsts = jnp.concatenate(dst_qs, axis=0)

    srcs1 = _pad_chunks(src, 0)
    srcs2 = jnp.concatenate([srcs1, srcs1 + N_NODES], axis=0)

    cut = N_EDGES - jnp.sum(key)
    hi0 = jnp.clip(cut - jnp.arange(32) * EPT, 0, EPT).astype(jnp.int32)
    nhi = (hi0 + CHUNK - 1) // CHUNK
    flo = hi0 // CHUNK
    np0 = (nhi + 1) // 2
    np1 = (NCHUNK - flo + 1) // 2
    bounds = jnp.zeros((64, 16), jnp.int32)
    bounds = bounds.at[:32, 1].set(np0)
    bounds = bounds.at[32:, 0].set(flo)
    bounds = bounds.at[32:, 1].set(np1)

    zeros = jnp.zeros((ARPT, 128), jnp.float32)
    agg2 = _make_sc_agg(N_NODES, [(0, 0), (1, 0)])
    agg4 = _make_sc_agg(2 * N_NODES, [(0, 0), (0, 1), (1, 0), (1, 1)])

    z = x
    outs = []
    for p in params:
        d = z.shape[1]
        if d == 128:
            agg = agg2(z, srcs1, dsts, bounds, zeros)
            z = _mlp(z, agg, p, 2)
        else:
            dh = d // 2
            z_stack = jnp.concatenate([z[:, :dh], z[:, dh:]], axis=0)
            agg = agg4(z_stack, srcs2, dsts, bounds, zeros)
            z = _mlp(z, agg, p, 4)
        outs.append(z)
    return jnp.concatenate(outs, axis=1)
